# Initial kernel scaffold; baseline (speedup 1.0000x reference)
import jax
import jax.numpy as jnp
from jax.experimental import pallas as pl


def kernel(colors, weights, min, max, ray_indices, num_rays):
    raise NotImplementedError("write your pallas kernel here")


# speedup vs baseline: 2.7505x; 2.7505x over previous
import functools

import jax
import jax.numpy as jnp
from jax import lax
from jax.experimental import pallas as pl
from jax.experimental.pallas import tpu as pltpu
from jax.experimental.pallas import tpu_sc as plsc

N_SAMPLES = 3145728
NUM_RAYS_C = 65536
NC = 2
NS = 16
NW = NC * NS
RPT = NUM_RAYS_C // NW
OUTW = RPT * 3
L = 16
CHUNK = 4096
G16 = N_SAMPLES // L
SEARCH_ITERS = 18


def _sc_body(colors_hbm, weights_hbm, rays_hbm, lo16_hbm, hi16_hbm, out_hbm,
             ray_v, w_v, c_v, acc_v, probe_v, clip_v, sem0, sem1, sem2):
    cid = lax.axis_index("c")
    sid = lax.axis_index("s")
    wid = cid * NS + sid
    base = wid * RPT

    def find_boundary(target):
        def it(_, carry):
            lo, hi = carry
            mid = jnp.minimum((lo + hi) // 2, G16 - 1)
            pltpu.sync_copy(rays_hbm.at[pl.ds(pl.multiple_of(mid * L, L), L)],
                            probe_v)
            go_left = probe_v[...][0] >= target
            return (jnp.where(go_left, lo, mid + 1),
                    jnp.where(go_left, mid, hi))
        lo, _ = lax.fori_loop(0, SEARCH_ITERS, it,
                              (jnp.int32(0), jnp.int32(G16)))
        return lo

    lo_s = jnp.maximum(find_boundary(base) - 1, 0) * L
    hi_s = find_boundary(base + RPT) * L

    zeros16 = jnp.zeros((L,), jnp.float32)

    def zero_it(j, _):
        acc_v[pl.ds(j * L, L)] = zeros16
        return 0
    lax.fori_loop(0, OUTW // L, zero_it, 0)

    iota = lax.iota(jnp.int32, L)
    gk = [(iota + L * k) // 3 for k in range(3)]
    moff = [(iota + L * k) % 3 - 3 * base for k in range(3)]

    lo_a = (lo_s // L) * L
    nsamp = hi_s - lo_a
    nchunks = (nsamp + CHUNK - 1) // CHUNK

    def chunk_body(ci, _):
        start_nom = lo_a + ci * CHUNK
        start = pl.multiple_of(jnp.minimum(start_nom, N_SAMPLES - CHUNK), L)
        cp0 = pltpu.async_copy(rays_hbm.at[pl.ds(start, CHUNK)], ray_v, sem0)
        cp1 = pltpu.async_copy(weights_hbm.at[pl.ds(start, CHUNK)], w_v, sem1)
        cp2 = pltpu.async_copy(
            colors_hbm.at[pl.ds(pl.multiple_of(start * 3, 8), 3 * CHUNK)],
            c_v, sem2)
        cp0.wait()
        cp1.wait()
        cp2.wait()
        a_s = jnp.maximum(lo_s, start_nom) - start
        b_s = jnp.minimum(hi_s, start_nom + CHUNK) - start

        def group_body(i, _):
            off = i * L
            for k in range(3):
                p_k = gk[k] + off
                rk = plsc.load_gather(ray_v, [p_k])
                wk = plsc.load_gather(w_v, [p_k])
                ck = c_v[pl.ds(off * 3 + k * L, L)]
                tk = rk * 3 + moff[k]
                mask = ((p_k >= a_s) & (p_k < b_s)
                        & (tk >= 0) & (tk < OUTW))
                plsc.addupdate_scatter(acc_v, [tk], ck * wk, mask=mask)
            return 0
        lax.fori_loop(0, CHUNK // L, group_body, 0)
        return 0

    lax.fori_loop(0, nchunks, chunk_body, 0)

    pltpu.sync_copy(lo16_hbm, clip_v)
    lo_vec = clip_v[...]
    pltpu.sync_copy(hi16_hbm, clip_v)
    hi_vec = clip_v[...]

    def clip_it(j, _):
        v = acc_v[pl.ds(j * L, L)]
        acc_v[pl.ds(j * L, L)] = jnp.minimum(jnp.maximum(v, lo_vec), hi_vec)
        return 0
    lax.fori_loop(0, OUTW // L, clip_it, 0)

    pltpu.sync_copy(acc_v, out_hbm.at[pl.ds(wid * OUTW, OUTW)])


@functools.partial(
    pl.kernel,
    out_type=jax.ShapeDtypeStruct((NUM_RAYS_C * 3,), jnp.float32),
    mesh=plsc.VectorSubcoreMesh(core_axis_name="c", subcore_axis_name="s",
                                num_cores=NC, num_subcores=NS),
    compiler_params=pltpu.CompilerParams(needs_layout_passes=False),
    scratch_types=[
        pltpu.VMEM((CHUNK,), jnp.int32),
        pltpu.VMEM((CHUNK,), jnp.float32),
        pltpu.VMEM((3 * CHUNK,), jnp.float32),
        pltpu.VMEM((OUTW,), jnp.float32),
        pltpu.VMEM((L,), jnp.int32),
        pltpu.VMEM((L,), jnp.float32),
        pltpu.SemaphoreType.DMA,
        pltpu.SemaphoreType.DMA,
        pltpu.SemaphoreType.DMA,
    ],
)
def _sc_segment_rgb(colors_hbm, weights_hbm, rays_hbm, lo16_hbm, hi16_hbm,
                    out_hbm, ray_v, w_v, c_v, acc_v, probe_v, clip_v,
                    sem0, sem1, sem2):
    _sc_body(colors_hbm, weights_hbm, rays_hbm, lo16_hbm, hi16_hbm, out_hbm,
             ray_v, w_v, c_v, acc_v, probe_v, clip_v, sem0, sem1, sem2)


def kernel(colors, weights, min, max, ray_indices, num_rays):
    cf = colors.reshape(-1)
    wf = weights.reshape(-1)
    ri = ray_indices.astype(jnp.int32)
    lo16 = jnp.full((L,), min, dtype=jnp.float32)
    hi16 = jnp.full((L,), max, dtype=jnp.float32)
    out = _sc_segment_rgb(cf, wf, ri, lo16, hi16)
    return out.reshape(NUM_RAYS_C, 3)


# speedup vs baseline: 2.7791x; 1.0104x over previous
import functools

import jax
import jax.numpy as jnp
from jax import lax
from jax.experimental import pallas as pl
from jax.experimental.pallas import tpu as pltpu
from jax.experimental.pallas import tpu_sc as plsc

N_SAMPLES = 3145728
NUM_RAYS_C = 65536
NC = 2
NS = 16
NW = NC * NS
RPT = NUM_RAYS_C // NW
OUTW = RPT * 3
L = 16
CHUNK = 4096
G16 = N_SAMPLES // L
SEARCH_ITERS = 18


def _sc_body(colors_hbm, weights_hbm, rays_hbm, lo16_hbm, hi16_hbm, out_hbm,
             ray_v, w_v, c_v, acc_v, probe_v, clip_v,
             sem0, sem1, sem2, sem3, sem4, sem5):
    cid = lax.axis_index("c")
    sid = lax.axis_index("s")
    wid = cid * NS + sid
    base = wid * RPT

    def find_boundary(target):
        def it(_, carry):
            lo, hi = carry
            mid = jnp.minimum((lo + hi) // 2, G16 - 1)
            pltpu.sync_copy(rays_hbm.at[pl.ds(pl.multiple_of(mid * L, L), L)],
                            probe_v)
            go_left = probe_v[...][0] >= target
            return (jnp.where(go_left, lo, mid + 1),
                    jnp.where(go_left, mid, hi))
        lo, _ = lax.fori_loop(0, SEARCH_ITERS, it,
                              (jnp.int32(0), jnp.int32(G16)))
        return lo

    lo_s = jnp.maximum(find_boundary(base) - 1, 0) * L
    hi_s = find_boundary(base + RPT) * L

    zeros16 = jnp.zeros((L,), jnp.float32)

    def zero_it(j, _):
        acc_v[pl.ds(j * L, L)] = zeros16
        return 0
    lax.fori_loop(0, OUTW // L, zero_it, 0)

    iota = lax.iota(jnp.int32, L)
    gk = [(iota + L * k) // 3 for k in range(3)]
    moff = [(iota + L * k) % 3 - 3 * base for k in range(3)]

    lo_a = (lo_s // L) * L
    nsamp = hi_s - lo_a
    nchunks = (nsamp + CHUNK - 1) // CHUNK

    sems = ((sem0, sem1, sem2), (sem3, sem4, sem5))

    def chunk_start(ci):
        start_nom = lo_a + ci * CHUNK
        start = pl.multiple_of(jnp.minimum(start_nom, N_SAMPLES - CHUNK), L)
        return start_nom, start

    def copies(ci, par):
        _, start = chunk_start(ci)
        boff = par * CHUNK
        sr, sw, sc = sems[par]
        return (
            pltpu.make_async_copy(rays_hbm.at[pl.ds(start, CHUNK)],
                                  ray_v.at[pl.ds(boff, CHUNK)], sr),
            pltpu.make_async_copy(weights_hbm.at[pl.ds(start, CHUNK)],
                                  w_v.at[pl.ds(boff, CHUNK)], sw),
            pltpu.make_async_copy(
                colors_hbm.at[pl.ds(pl.multiple_of(start * 3, 8), 3 * CHUNK)],
                c_v.at[pl.ds(3 * boff, 3 * CHUNK)], sc),
        )

    def issue(ci, par):
        for cp in copies(ci, par):
            cp.start()

    def compute(ci, par):
        boff = par * CHUNK
        start_nom, start = chunk_start(ci)
        a_s = jnp.maximum(lo_s, start_nom) - start + boff
        b_s = jnp.minimum(hi_s, start_nom + CHUNK) - start + boff

        def group_body(i, _):
            off = i * L + boff
            for k in range(3):
                p_k = gk[k] + off
                rk = plsc.load_gather(ray_v, [p_k])
                wk = plsc.load_gather(w_v, [p_k])
                ck = c_v[pl.ds(off * 3 + k * L, L)]
                tk = rk * 3 + moff[k]
                mask = ((p_k >= a_s) & (p_k < b_s)
                        & (tk >= 0) & (tk < OUTW))
                plsc.addupdate_scatter(acc_v, [tk], ck * wk, mask=mask)
            return 0
        lax.fori_loop(0, CHUNK // L, group_body, 0, unroll=8)

    @pl.when(nchunks > 0)
    def _prime():
        issue(0, 0)

    def pair_body(c, _):
        for b in range(2):
            ci = 2 * c + b

            @pl.when(ci < nchunks)
            def _step():
                for cp in copies(ci, b):
                    cp.wait()

                @pl.when(ci + 1 < nchunks)
                def _prefetch():
                    issue(ci + 1, 1 - b)

                compute(ci, b)
        return 0

    lax.fori_loop(0, (nchunks + 1) // 2, pair_body, 0)

    pltpu.sync_copy(lo16_hbm, clip_v)
    lo_vec = clip_v[...]
    pltpu.sync_copy(hi16_hbm, clip_v)
    hi_vec = clip_v[...]

    def clip_it(j, _):
        v = acc_v[pl.ds(j * L, L)]
        acc_v[pl.ds(j * L, L)] = jnp.minimum(jnp.maximum(v, lo_vec), hi_vec)
        return 0
    lax.fori_loop(0, OUTW // L, clip_it, 0)

    pltpu.sync_copy(acc_v, out_hbm.at[pl.ds(wid * OUTW, OUTW)])


@functools.partial(
    pl.kernel,
    out_type=jax.ShapeDtypeStruct((NUM_RAYS_C * 3,), jnp.float32),
    mesh=plsc.VectorSubcoreMesh(core_axis_name="c", subcore_axis_name="s",
                                num_cores=NC, num_subcores=NS),
    compiler_params=pltpu.CompilerParams(needs_layout_passes=False),
    scratch_types=[
        pltpu.VMEM((2 * CHUNK,), jnp.int32),
        pltpu.VMEM((2 * CHUNK,), jnp.float32),
        pltpu.VMEM((6 * CHUNK,), jnp.float32),
        pltpu.VMEM((OUTW,), jnp.float32),
        pltpu.VMEM((L,), jnp.int32),
        pltpu.VMEM((L,), jnp.float32),
        pltpu.SemaphoreType.DMA,
        pltpu.SemaphoreType.DMA,
        pltpu.SemaphoreType.DMA,
        pltpu.SemaphoreType.DMA,
        pltpu.SemaphoreType.DMA,
        pltpu.SemaphoreType.DMA,
    ],
)
def _sc_segment_rgb(colors_hbm, weights_hbm, rays_hbm, lo16_hbm, hi16_hbm,
                    out_hbm, ray_v, w_v, c_v, acc_v, probe_v, clip_v,
                    sem0, sem1, sem2, sem3, sem4, sem5):
    _sc_body(colors_hbm, weights_hbm, rays_hbm, lo16_hbm, hi16_hbm, out_hbm,
             ray_v, w_v, c_v, acc_v, probe_v, clip_v,
             sem0, sem1, sem2, sem3, sem4, sem5)


def kernel(colors, weights, min, max, ray_indices, num_rays):
    cf = colors.reshape(-1)
    wf = weights.reshape(-1)
    ri = ray_indices.astype(jnp.int32)
    lo16 = jnp.full((L,), min, dtype=jnp.float32)
    hi16 = jnp.full((L,), max, dtype=jnp.float32)
    out = _sc_segment_rgb(cf, wf, ri, lo16, hi16)
    return out.reshape(NUM_RAYS_C, 3)


# speedup vs baseline: 23.7885x; 8.5597x over previous
import functools

import jax
import jax.numpy as jnp
from jax import lax
from jax.experimental import pallas as pl
from jax.experimental.pallas import tpu as pltpu
from jax.experimental.pallas import tpu_sc as plsc

N_SAMPLES = 3145728
NUM_RAYS_C = 65536
NC = 2
NS = 16
NW = NC * NS
RPT = NUM_RAYS_C // NW
OUTW = RPT * 3
L = 16
B = 128
CHUNK = 4096
G16 = N_SAMPLES // L
SEARCH_ITERS = 18


def _sc_body(colors_hbm, weights_hbm, rays_hbm, lo16_hbm, hi16_hbm, out_hbm,
             ray_v, w_v, c_v, acc_v, probe_v, clip_v,
             sem0, sem1, sem2, sem3, sem4, sem5):
    cid = lax.axis_index("c")
    sid = lax.axis_index("s")
    wid = cid * NS + sid
    base = wid * RPT

    def find_cover(target):
        def it(_, carry):
            lo, hi = carry
            mid = jnp.minimum((lo + hi) // 2, G16 - 1)
            pltpu.sync_copy(rays_hbm.at[pl.ds(pl.multiple_of(mid * L, L), L)],
                            probe_v)
            go_left = probe_v[...][0] >= target
            return (jnp.where(go_left, lo, mid + 1),
                    jnp.where(go_left, mid, hi))
        lo, _ = lax.fori_loop(0, SEARCH_ITERS, it,
                              (jnp.int32(0), jnp.int32(G16)))
        return lo

    lo_s = jnp.maximum(find_cover(base) - 1, 0) * L
    hi_s = find_cover(base + RPT) * L

    zeros16 = jnp.zeros((L,), jnp.float32)

    def zero_it(j, _):
        acc_v[pl.ds(j * L, L)] = zeros16
        return 0
    lax.fori_loop(0, OUTW // L, zero_it, 0)

    iota = lax.iota(jnp.int32, L)

    lo_a = (lo_s // B) * B
    nsamp = hi_s - lo_a
    nchunks = (nsamp + CHUNK - 1) // CHUNK

    sems = ((sem0, sem1, sem2), (sem3, sem4, sem5))

    def chunk_start(ci):
        start_nom = lo_a + ci * CHUNK
        start = pl.multiple_of(jnp.minimum(start_nom, N_SAMPLES - CHUNK), B)
        return start_nom, start

    def copies(ci, par):
        _, start = chunk_start(ci)
        boff = par * CHUNK
        sr, sw, sc = sems[par]
        return (
            pltpu.make_async_copy(rays_hbm.at[pl.ds(start, CHUNK)],
                                  ray_v.at[pl.ds(boff, CHUNK)], sr),
            pltpu.make_async_copy(weights_hbm.at[pl.ds(start, CHUNK)],
                                  w_v.at[pl.ds(boff, CHUNK)], sw),
            pltpu.make_async_copy(
                colors_hbm.at[pl.ds(pl.multiple_of(start * 3, 8), 3 * CHUNK)],
                c_v.at[pl.ds(3 * boff, 3 * CHUNK)], sc),
        )

    def issue(ci, par):
        for cp in copies(ci, par):
            cp.start()

    def compute(ci, par):
        boff = par * CHUNK
        start_nom, start = chunk_start(ci)
        a_s = start_nom - start + boff

        def block_body(ob, _):
            rbase = boff + ob * B
            cbase = 3 * boff + ob * (3 * B)
            for g in range(B // L):
                roff = rbase + g * L
                rv = ray_v[pl.ds(roff, L)]
                wv = w_v[pl.ds(roff, L)]
                dv = rv - base
                du = plsc.bitcast(dv, jnp.uint32)
                mask = (du < jnp.uint32(RPT)) & (iota >= a_s - roff)
                t0 = dv * 3
                for ch in range(3):
                    cv = c_v[pl.ds(cbase + ch * B + g * L, L)]
                    tk = t0 if ch == 0 else t0 + ch
                    plsc.addupdate_scatter(acc_v, [tk], cv * wv, mask=mask)
            return 0
        lax.fori_loop(0, CHUNK // B, block_body, 0, unroll=2)

    @pl.when(nchunks > 0)
    def _prime():
        issue(0, 0)

    def pair_body(c, _):
        for b in range(2):
            ci = 2 * c + b

            @pl.when(ci < nchunks)
            def _step():
                for cp in copies(ci, b):
                    cp.wait()

                @pl.when(ci + 1 < nchunks)
                def _prefetch():
                    issue(ci + 1, 1 - b)

                compute(ci, b)
        return 0

    lax.fori_loop(0, (nchunks + 1) // 2, pair_body, 0)

    pltpu.sync_copy(lo16_hbm, clip_v)
    lo_vec = clip_v[...]
    pltpu.sync_copy(hi16_hbm, clip_v)
    hi_vec = clip_v[...]

    def clip_it(j, _):
        v = acc_v[pl.ds(j * L, L)]
        acc_v[pl.ds(j * L, L)] = jnp.minimum(jnp.maximum(v, lo_vec), hi_vec)
        return 0
    lax.fori_loop(0, OUTW // L, clip_it, 0)

    pltpu.sync_copy(acc_v, out_hbm.at[pl.ds(wid * OUTW, OUTW)])


@functools.partial(
    pl.kernel,
    out_type=jax.ShapeDtypeStruct((NUM_RAYS_C * 3,), jnp.float32),
    mesh=plsc.VectorSubcoreMesh(core_axis_name="c", subcore_axis_name="s",
                                num_cores=NC, num_subcores=NS),
    compiler_params=pltpu.CompilerParams(needs_layout_passes=False),
    scratch_types=[
        pltpu.VMEM((2 * CHUNK,), jnp.int32),
        pltpu.VMEM((2 * CHUNK,), jnp.float32),
        pltpu.VMEM((6 * CHUNK,), jnp.float32),
        pltpu.VMEM((OUTW,), jnp.float32),
        pltpu.VMEM((L,), jnp.int32),
        pltpu.VMEM((L,), jnp.float32),
        pltpu.SemaphoreType.DMA,
        pltpu.SemaphoreType.DMA,
        pltpu.SemaphoreType.DMA,
        pltpu.SemaphoreType.DMA,
        pltpu.SemaphoreType.DMA,
        pltpu.SemaphoreType.DMA,
    ],
)
def _sc_segment_rgb(colors_hbm, weights_hbm, rays_hbm, lo16_hbm, hi16_hbm,
                    out_hbm, ray_v, w_v, c_v, acc_v, probe_v, clip_v,
                    sem0, sem1, sem2, sem3, sem4, sem5):
    _sc_body(colors_hbm, weights_hbm, rays_hbm, lo16_hbm, hi16_hbm, out_hbm,
             ray_v, w_v, c_v, acc_v, probe_v, clip_v,
             sem0, sem1, sem2, sem3, sem4, sem5)


def kernel(colors, weights, min, max, ray_indices, num_rays):
    cpl = (colors.reshape(N_SAMPLES // B, B, 3)
           .transpose(0, 2, 1)
           .reshape(3 * N_SAMPLES))
    wf = weights.reshape(-1)
    ri = ray_indices.astype(jnp.int32)
    lo16 = jnp.full((L,), min, dtype=jnp.float32)
    hi16 = jnp.full((L,), max, dtype=jnp.float32)
    out = _sc_segment_rgb(cpl, wf, ri, lo16, hi16)
    return out.reshape(NUM_RAYS_C, 3)


# speedup vs baseline: 43.5426x; 1.8304x over previous
import functools

import jax
import jax.numpy as jnp
from jax import lax
from jax.experimental import pallas as pl
from jax.experimental.pallas import tpu as pltpu
from jax.experimental.pallas import tpu_sc as plsc

N_SAMPLES = 3145728
NUM_RAYS_C = 65536
NC = 2
NS = 16
NW = NC * NS
RPT = NUM_RAYS_C // NW
OUTW = RPT * 3
L = 16
B = 128
CHUNK = 4096
G16 = N_SAMPLES // L
SEARCH_ITERS = 18


def _sc_body(colors_hbm, weights_hbm, rays_hbm, lo16_hbm, hi16_hbm, out_hbm,
             ray_v, w_v, c_v, acc_v, probe_v, clip_v,
             sem0, sem1, sem2, sem3, sem4, sem5):
    cid = lax.axis_index("c")
    sid = lax.axis_index("s")
    wid = cid * NS + sid
    base = wid * RPT

    def find_cover(target):
        def it(_, carry):
            lo, hi = carry
            mid = jnp.minimum((lo + hi) // 2, G16 - 1)
            pltpu.sync_copy(rays_hbm.at[pl.ds(pl.multiple_of(mid * L, L), L)],
                            probe_v)
            go_left = probe_v[...][0] >= target
            return (jnp.where(go_left, lo, mid + 1),
                    jnp.where(go_left, mid, hi))
        lo, _ = lax.fori_loop(0, SEARCH_ITERS, it,
                              (jnp.int32(0), jnp.int32(G16)))
        return lo

    lo_s = jnp.maximum(find_cover(base) - 1, 0) * L
    hi_s = find_cover(base + RPT) * L

    zeros16 = jnp.zeros((L,), jnp.float32)

    def zero_it(j, _):
        acc_v[pl.ds(j * L, L)] = zeros16
        return 0
    lax.fori_loop(0, OUTW // L, zero_it, 0)

    iota = lax.iota(jnp.int32, L)

    lo_a = (lo_s // B) * B
    nsamp = hi_s - lo_a
    nchunks = (nsamp + CHUNK - 1) // CHUNK

    sems = ((sem0, sem1, sem2), (sem3, sem4, sem5))

    def chunk_start(ci):
        start_nom = lo_a + ci * CHUNK
        start = pl.multiple_of(jnp.minimum(start_nom, N_SAMPLES - CHUNK), B)
        return start_nom, start

    def copies(ci, par):
        _, start = chunk_start(ci)
        boff = par * CHUNK
        sr, sw, sc = sems[par]
        return (
            pltpu.make_async_copy(rays_hbm.at[pl.ds(start, CHUNK)],
                                  ray_v.at[pl.ds(boff, CHUNK)], sr),
            pltpu.make_async_copy(weights_hbm.at[pl.ds(start, CHUNK)],
                                  w_v.at[pl.ds(boff, CHUNK)], sw),
            pltpu.make_async_copy(
                colors_hbm.at[pl.ds(pl.multiple_of(start * 3, 8), 3 * CHUNK)],
                c_v.at[pl.ds(3 * boff, 3 * CHUNK)], sc),
        )

    def issue(ci, par):
        for cp in copies(ci, par):
            cp.start()

    SUB = CHUNK // L

    def compute(ci, par):
        boff = par * CHUNK
        start_nom, start = chunk_start(ci)
        a_s = start_nom - start + boff
        lane_base = iota * SUB + boff

        def step(i, _):
            u = (iota + i) & (SUB - 1)
            sv = lane_base + u
            rv = plsc.load_gather(ray_v, [sv])
            wv = plsc.load_gather(w_v, [sv])
            dv = rv - base
            du = plsc.bitcast(dv, jnp.uint32)
            mask = (du < jnp.uint32(RPT)) & (sv >= a_s)
            t0 = dv * 3
            cb = (sv >> 7) * (3 * B) + (sv & (B - 1))
            for ch in range(3):
                cv = plsc.load_gather(c_v, [cb if ch == 0 else cb + ch * B])
                tk = t0 if ch == 0 else t0 + ch
                plsc.addupdate_scatter(acc_v, [tk], cv * wv, mask=mask)
            return 0
        lax.fori_loop(0, SUB, step, 0, unroll=4)

    @pl.when(nchunks > 0)
    def _prime():
        issue(0, 0)

    def pair_body(c, _):
        for b in range(2):
            ci = 2 * c + b

            @pl.when(ci < nchunks)
            def _step():
                for cp in copies(ci, b):
                    cp.wait()

                @pl.when(ci + 1 < nchunks)
                def _prefetch():
                    issue(ci + 1, 1 - b)

                compute(ci, b)
        return 0

    lax.fori_loop(0, (nchunks + 1) // 2, pair_body, 0)

    pltpu.sync_copy(lo16_hbm, clip_v)
    lo_vec = clip_v[...]
    pltpu.sync_copy(hi16_hbm, clip_v)
    hi_vec = clip_v[...]

    def clip_it(j, _):
        v = acc_v[pl.ds(j * L, L)]
        acc_v[pl.ds(j * L, L)] = jnp.minimum(jnp.maximum(v, lo_vec), hi_vec)
        return 0
    lax.fori_loop(0, OUTW // L, clip_it, 0)

    pltpu.sync_copy(acc_v, out_hbm.at[pl.ds(wid * OUTW, OUTW)])


@functools.partial(
    pl.kernel,
    out_type=jax.ShapeDtypeStruct((NUM_RAYS_C * 3,), jnp.float32),
    mesh=plsc.VectorSubcoreMesh(core_axis_name="c", subcore_axis_name="s",
                                num_cores=NC, num_subcores=NS),
    compiler_params=pltpu.CompilerParams(needs_layout_passes=False),
    scratch_types=[
        pltpu.VMEM((2 * CHUNK,), jnp.int32),
        pltpu.VMEM((2 * CHUNK,), jnp.float32),
        pltpu.VMEM((6 * CHUNK,), jnp.float32),
        pltpu.VMEM((OUTW,), jnp.float32),
        pltpu.VMEM((L,), jnp.int32),
        pltpu.VMEM((L,), jnp.float32),
        pltpu.SemaphoreType.DMA,
        pltpu.SemaphoreType.DMA,
        pltpu.SemaphoreType.DMA,
        pltpu.SemaphoreType.DMA,
        pltpu.SemaphoreType.DMA,
        pltpu.SemaphoreType.DMA,
    ],
)
def _sc_segment_rgb(colors_hbm, weights_hbm, rays_hbm, lo16_hbm, hi16_hbm,
                    out_hbm, ray_v, w_v, c_v, acc_v, probe_v, clip_v,
                    sem0, sem1, sem2, sem3, sem4, sem5):
    _sc_body(colors_hbm, weights_hbm, rays_hbm, lo16_hbm, hi16_hbm, out_hbm,
             ray_v, w_v, c_v, acc_v, probe_v, clip_v,
             sem0, sem1, sem2, sem3, sem4, sem5)


def kernel(colors, weights, min, max, ray_indices, num_rays):
    cpl = (colors.reshape(N_SAMPLES // B, B, 3)
           .transpose(0, 2, 1)
           .reshape(3 * N_SAMPLES))
    wf = weights.reshape(-1)
    ri = ray_indices.astype(jnp.int32)
    lo16 = jnp.full((L,), min, dtype=jnp.float32)
    hi16 = jnp.full((L,), max, dtype=jnp.float32)
    out = _sc_segment_rgb(cpl, wf, ri, lo16, hi16)
    return out.reshape(NUM_RAYS_C, 3)
